# Initial kernel scaffold; baseline (speedup 1.0000x reference)
import jax
import jax.numpy as jnp
from jax.experimental import pallas as pl


def kernel(x_bx2, opacity, rgbsh, mu, scale, angle):
    raise NotImplementedError("write your pallas kernel here")


# speedup vs baseline: 2053.6547x; 2053.6547x over previous
import jax
import jax.numpy as jnp
from jax.experimental import pallas as pl
from jax.experimental.pallas import tpu as pltpu

H = 512
W = 512
NG = 10000
NSH = 4
MU_BORDER = 1.05
S_MIN = 1.0 / 30.0
S_MAX = 1.0 / 0.75
GBLK = 128
NPAD = 10240
NSTEP = NPAD // GBLK


def _body(x_ref, p_ref, sh_ref, o_ref, acc0, acc1, acc2):
    pid = pl.program_id(0)
    gmu0 = jnp.tanh(p_ref[0:1, :]) * MU_BORDER
    gmu1 = jnp.tanh(p_ref[1:2, :]) * MU_BORDER
    s0 = jnp.clip(p_ref[2:3, :], 0.0, 1.0) * (S_MAX - S_MIN) + S_MIN
    s1g = jnp.clip(p_ref[3:4, :], 0.0, 1.0) * (S_MAX - S_MIN) + S_MIN
    alpha = jnp.tanh(p_ref[4:5, :]) * 3.1416
    ca = jnp.cos(alpha)
    sa = jnp.sin(alpha)
    opg = jax.nn.sigmoid(p_ref[5:6, :])

    x0 = x_ref[:, 0:1] * (2.0 / W) - 1.0
    x1 = x_ref[:, 1:2] * (2.0 / H) - 1.0

    v0 = (x0 - gmu0) * (0.5 * W)
    v1 = (x1 - gmu1) * (0.5 * H)
    a0 = s0 * v0
    a1 = s1g * v1
    d2 = a0 * a0 + a1 * a1
    w = jnp.where(d2 < 25.0, jnp.exp(-d2), 0.0) * opg

    inv = 1.0 / (1e-10 + jnp.sqrt(v0 * v0 + v1 * v1))
    vn0 = v0 * inv
    vn1 = v1 * inv
    sin1 = ca * vn0 - sa * vn1
    cos1 = sa * vn0 + ca * vn1
    sin2 = sin1 * cos1 + cos1 * sin1
    cos2 = cos1 * cos1 - sin1 * sin1
    sin3 = sin2 * cos1 + cos2 * sin1
    cos3 = cos2 * cos1 - sin2 * sin1
    sin4 = sin3 * cos1 + cos3 * sin1
    cos4 = cos3 * cos1 - sin3 * sin1

    accs = (acc0, acc1, acc2)
    for ch in range(3):
        sh = lambda m: sh_ref[3 * m + ch:3 * m + ch + 1, :]
        t = (sh(0) + sin1 * sh(1) + cos1 * sh(2) + sin2 * sh(3)
             + cos2 * sh(4) + sin3 * sh(5) + cos3 * sh(6)
             + sin4 * sh(7) + cos4 * sh(8))
        contrib = w * jax.nn.sigmoid(t)
        aref = accs[ch]

        @pl.when(pid == 0)
        def _():
            aref[...] = contrib

        @pl.when(pid != 0)
        def _():
            aref[...] += contrib

    @pl.when(pid == NSTEP - 1)
    def _():
        r0 = jnp.sum(acc0[...], axis=1, keepdims=True)
        r1 = jnp.sum(acc1[...], axis=1, keepdims=True)
        r2 = jnp.sum(acc2[...], axis=1, keepdims=True)
        o_ref[...] = jnp.concatenate([r0, r1, r2], axis=1)


def kernel(x_bx2, opacity, rgbsh, mu, scale, angle):
    b = x_bx2.shape[0]
    pad = NPAD - NG
    mu0p = jnp.concatenate([mu[:, 0], jnp.full((pad,), 37.0, jnp.float32)])
    mu1p = jnp.concatenate([mu[:, 1], jnp.zeros((pad,), jnp.float32)])
    sc0p = jnp.concatenate([scale[:, 0], jnp.ones((pad,), jnp.float32)])
    sc1p = jnp.concatenate([scale[:, 1], jnp.ones((pad,), jnp.float32)])
    angp = jnp.concatenate([angle, jnp.zeros((pad,), jnp.float32)])
    opp = jnp.concatenate([opacity, jnp.zeros((pad,), jnp.float32)])
    zp = jnp.zeros((NPAD,), jnp.float32)
    p_all = jnp.stack([mu0p, mu1p, sc0p, sc1p, angp, opp, zp, zp], axis=0)

    sh_t = rgbsh.reshape(NG, 27).T
    sh_t = jnp.concatenate([sh_t, jnp.zeros((27, pad), jnp.float32)], axis=1)
    sh_t = jnp.concatenate([sh_t, jnp.zeros((5, NPAD), jnp.float32)], axis=0)

    out = pl.pallas_call(
        _body,
        grid=(NSTEP,),
        in_specs=[
            pl.BlockSpec((b, 2), lambda i: (0, 0)),
            pl.BlockSpec((8, GBLK), lambda i: (0, i)),
            pl.BlockSpec((32, GBLK), lambda i: (0, i)),
        ],
        out_specs=pl.BlockSpec((b, 3), lambda i: (0, 0)),
        out_shape=jax.ShapeDtypeStruct((b, 3), jnp.float32),
        scratch_shapes=[
            pltpu.VMEM((b, GBLK), jnp.float32),
            pltpu.VMEM((b, GBLK), jnp.float32),
            pltpu.VMEM((b, GBLK), jnp.float32),
        ],
        compiler_params=pltpu.CompilerParams(
            dimension_semantics=("arbitrary",),
        ),
    )(x_bx2, p_all, sh_t)
    return out
